# baseline (speedup 1.0000x reference)
import functools

import jax
import jax.numpy as jnp
from jax import lax
from jax.experimental import pallas as pl
from jax.experimental.pallas import tpu as pltpu
from jax.experimental.pallas import tpu_sc as plsc

_B = 4
_S = 2048
_D = 768
_EPS = 1e-12

_N_TOK = _B * _S
_NC = 4
_PCH = _S // _NC
_NCHTOK = _N_TOK // _NC
_NW = 32
_WSEG = _NW // _B
_TPW = _PCH // _WSEG
_K = 32
_NGR = _TPW // _K


def _gather_body(chunk, ids_hbm, wemb_hbm, out_hbm,
                 idx0, idx1, row0, row1, sg0, sg1, ss0, ss1):
    cid = lax.axis_index("c")
    sid = lax.axis_index("s")
    w = sid * 2 + cid
    seg = w // _WSEG
    off = (w % _WSEG) * _TPW
    pbase = chunk * _PCH + off
    obase = seg * _PCH + off
    idx = (idx0, idx1)
    row = (row0, row1)
    sg = (sg0, sg1)
    ss = (ss0, ss1)

    pltpu.sync_copy(ids_hbm.at[seg, pl.ds(pbase, _K)], idx0)
    pltpu.async_copy(wemb_hbm.at[idx0], row0, sg0)
    for c in range(_NGR):
        b = c & 1
        if c + 1 < _NGR:
            pltpu.sync_copy(ids_hbm.at[seg, pl.ds(pbase + (c + 1) * _K, _K)],
                            idx[1 - b])
            if c >= 1:
                pltpu.make_async_copy(
                    row[1 - b], out_hbm.at[pl.ds(obase + (c - 1) * _K, _K)],
                    ss[1 - b]).wait()
            pltpu.async_copy(wemb_hbm.at[idx[1 - b]], row[1 - b], sg[1 - b])
        pltpu.make_async_copy(wemb_hbm.at[idx[b]], row[b], sg[b]).wait()
        pltpu.async_copy(row[b], out_hbm.at[pl.ds(obase + c * _K, _K)], ss[b])
    for c in range(max(0, _NGR - 2), _NGR):
        b = c & 1
        pltpu.make_async_copy(
            row[b], out_hbm.at[pl.ds(obase + c * _K, _K)], ss[b]).wait()


def _sc_gather(ids, wemb, chunk):
    mesh = plsc.VectorSubcoreMesh(core_axis_name="c", subcore_axis_name="s")
    f = functools.partial(
        pl.kernel,
        mesh=mesh,
        compiler_params=pltpu.CompilerParams(needs_layout_passes=False),
        out_type=jax.ShapeDtypeStruct((_NCHTOK, _D), jnp.float32),
        scratch_types=[
            pltpu.VMEM((_K,), jnp.int32),
            pltpu.VMEM((_K,), jnp.int32),
            pltpu.VMEM((_K, _D), jnp.float32),
            pltpu.VMEM((_K, _D), jnp.float32),
            pltpu.SemaphoreType.DMA,
            pltpu.SemaphoreType.DMA,
            pltpu.SemaphoreType.DMA,
            pltpu.SemaphoreType.DMA,
        ],
    )(functools.partial(_gather_body, chunk))
    return f(ids, wemb)


def _ln_body(rows_ref, pos_ref, typ_ref, gam_ref, bet_ref, out_ref):
    x = rows_ref[...] + pos_ref[...] + typ_ref[...]
    mean = jnp.mean(x, axis=1, keepdims=True)
    xc = x - mean
    var = jnp.mean(xc * xc, axis=1, keepdims=True)
    y = xc * lax.rsqrt(var + _EPS)
    out_ref[...] = (y * gam_ref[...] + bet_ref[...]).astype(jnp.bfloat16)


def _ln_body_alias(prev_ref, rows_ref, pos_ref, typ_ref, gam_ref, bet_ref,
                   out_ref):
    del prev_ref
    _ln_body(rows_ref, pos_ref, typ_ref, gam_ref, bet_ref, out_ref)


def _ln_specs(chunk):
    return [
        pl.BlockSpec((_PCH, _D), lambda b: (b, 0)),
        pl.BlockSpec((_PCH, _D), lambda b: (chunk, 0)),
        pl.BlockSpec((1, _D), lambda b: (0, 0)),
        pl.BlockSpec((1, _D), lambda b: (0, 0)),
        pl.BlockSpec((1, _D), lambda b: (0, 0)),
    ]


def _ln_out_spec(chunk):
    return pl.BlockSpec((_PCH, _D), lambda b: (b * _NC + chunk, 0))


def _tc_ln_chunk(prev, rows, pos, typ0, gam2, bet2, chunk):
    out_shape = jax.ShapeDtypeStruct((_N_TOK, _D), jnp.bfloat16)
    if prev is None:
        return pl.pallas_call(
            _ln_body,
            grid=(_B,),
            in_specs=_ln_specs(chunk),
            out_specs=_ln_out_spec(chunk),
            out_shape=out_shape,
        )(rows, pos, typ0, gam2, bet2)
    return pl.pallas_call(
        _ln_body_alias,
        grid=(_B,),
        in_specs=[pl.BlockSpec(memory_space=pl.ANY)] + _ln_specs(chunk),
        out_specs=_ln_out_spec(chunk),
        out_shape=out_shape,
        input_output_aliases={0: 0},
    )(prev, rows, pos, typ0, gam2, bet2)


@jax.jit
def _run(input_ids, wemb, pemb, temb, gam, bet):
    ids = input_ids.astype(jnp.int32)
    typ0 = temb[0:1]
    gam2 = gam.reshape(1, _D)
    bet2 = bet.reshape(1, _D)
    rows = [_sc_gather(ids, wemb, c) for c in range(_NC)]
    buf = None
    for c in range(_NC):
        buf = _tc_ln_chunk(buf, rows[c], pemb, typ0, gam2, bet2, c)
    return buf.reshape(_B, _S, _D)


def kernel(input_ids, word_emb, pos_emb, type_emb, gamma, beta):
    return _run(input_ids, word_emb, pos_emb, type_emb, gamma, beta)


# speedup vs baseline: 1.0521x; 1.0521x over previous
import functools

import jax
import jax.numpy as jnp
from jax import lax
from jax.experimental import pallas as pl
from jax.experimental.pallas import tpu as pltpu
from jax.experimental.pallas import tpu_sc as plsc

_B = 4
_S = 2048
_D = 768
_EPS = 1e-12

_N_TOK = _B * _S
_PCHUNKS = (1024, 512, 512)
_PSTARTS = (0, 1024, 1536)
_NW = 32
_WSEG = _NW // _B
_K = 32


def _gather_body(pstart, plen, ids_hbm, wemb_hbm, out_hbm,
                 idx0, idx1, row0, row1, sg0, sg1, ss0, ss1):
    cid = lax.axis_index("c")
    sid = lax.axis_index("s")
    w = sid * 2 + cid
    seg = w // _WSEG
    tpw = plen // _WSEG
    ngr = tpw // _K
    off = (w % _WSEG) * tpw
    pbase = pstart + off
    obase = seg * plen + off
    idx = (idx0, idx1)
    row = (row0, row1)
    sg = (sg0, sg1)
    ss = (ss0, ss1)

    pltpu.sync_copy(ids_hbm.at[seg, pl.ds(pbase, _K)], idx0)
    pltpu.async_copy(wemb_hbm.at[idx0], row0, sg0)
    for c in range(ngr):
        b = c & 1
        if c + 1 < ngr:
            pltpu.sync_copy(ids_hbm.at[seg, pl.ds(pbase + (c + 1) * _K, _K)],
                            idx[1 - b])
            if c >= 1:
                pltpu.make_async_copy(
                    row[1 - b], out_hbm.at[pl.ds(obase + (c - 1) * _K, _K)],
                    ss[1 - b]).wait()
            pltpu.async_copy(wemb_hbm.at[idx[1 - b]], row[1 - b], sg[1 - b])
        pltpu.make_async_copy(wemb_hbm.at[idx[b]], row[b], sg[b]).wait()
        pltpu.async_copy(row[b], out_hbm.at[pl.ds(obase + c * _K, _K)], ss[b])
    for c in range(max(0, ngr - 2), ngr):
        b = c & 1
        pltpu.make_async_copy(
            row[b], out_hbm.at[pl.ds(obase + c * _K, _K)], ss[b]).wait()


def _sc_gather(ids, wemb, chunk):
    pstart = _PSTARTS[chunk]
    plen = _PCHUNKS[chunk]
    mesh = plsc.VectorSubcoreMesh(core_axis_name="c", subcore_axis_name="s")
    f = functools.partial(
        pl.kernel,
        mesh=mesh,
        compiler_params=pltpu.CompilerParams(needs_layout_passes=False),
        out_type=jax.ShapeDtypeStruct((_B * plen, _D), jnp.float32),
        scratch_types=[
            pltpu.VMEM((_K,), jnp.int32),
            pltpu.VMEM((_K,), jnp.int32),
            pltpu.VMEM((_K, _D), jnp.float32),
            pltpu.VMEM((_K, _D), jnp.float32),
            pltpu.SemaphoreType.DMA,
            pltpu.SemaphoreType.DMA,
            pltpu.SemaphoreType.DMA,
            pltpu.SemaphoreType.DMA,
        ],
    )(functools.partial(_gather_body, pstart, plen))
    return f(ids, wemb)


def _ln_body(rows_ref, pos_ref, typ_ref, gam_ref, bet_ref, out_ref):
    x = rows_ref[...] + pos_ref[...] + typ_ref[...]
    mean = jnp.mean(x, axis=1, keepdims=True)
    xc = x - mean
    var = jnp.mean(xc * xc, axis=1, keepdims=True)
    y = xc * lax.rsqrt(var + _EPS)
    out_ref[...] = (y * gam_ref[...] + bet_ref[...]).astype(jnp.bfloat16)


def _ln_body_alias(prev_ref, rows_ref, pos_ref, typ_ref, gam_ref, bet_ref,
                   out_ref):
    del prev_ref
    _ln_body(rows_ref, pos_ref, typ_ref, gam_ref, bet_ref, out_ref)


def _ln_specs(chunk):
    plen = _PCHUNKS[chunk]
    poff = _PSTARTS[chunk] // plen
    return [
        pl.BlockSpec((plen, _D), lambda b: (b, 0)),
        pl.BlockSpec((plen, _D), lambda b: (poff, 0)),
        pl.BlockSpec((1, _D), lambda b: (0, 0)),
        pl.BlockSpec((1, _D), lambda b: (0, 0)),
        pl.BlockSpec((1, _D), lambda b: (0, 0)),
    ]


def _ln_out_spec(chunk):
    plen = _PCHUNKS[chunk]
    boff = _PSTARTS[chunk] // plen
    nblk = _S // plen
    return pl.BlockSpec((plen, _D), lambda b: (b * nblk + boff, 0))


def _tc_ln_chunk(prev, rows, pos, typ0, gam2, bet2, chunk):
    out_shape = jax.ShapeDtypeStruct((_N_TOK, _D), jnp.bfloat16)
    if prev is None:
        return pl.pallas_call(
            _ln_body,
            grid=(_B,),
            in_specs=_ln_specs(chunk),
            out_specs=_ln_out_spec(chunk),
            out_shape=out_shape,
        )(rows, pos, typ0, gam2, bet2)
    return pl.pallas_call(
        _ln_body_alias,
        grid=(_B,),
        in_specs=[pl.BlockSpec(memory_space=pl.ANY)] + _ln_specs(chunk),
        out_specs=_ln_out_spec(chunk),
        out_shape=out_shape,
        input_output_aliases={0: 0},
    )(prev, rows, pos, typ0, gam2, bet2)


@jax.jit
def _run(input_ids, wemb, pemb, temb, gam, bet):
    ids = input_ids.astype(jnp.int32)
    typ0 = temb[0:1]
    gam2 = gam.reshape(1, _D)
    bet2 = bet.reshape(1, _D)
    rows = [_sc_gather(ids, wemb, c) for c in range(len(_PCHUNKS))]
    buf = None
    for c in range(len(_PCHUNKS)):
        buf = _tc_ln_chunk(buf, rows[c], pemb, typ0, gam2, bet2, c)
    return buf.reshape(_B, _S, _D)


def kernel(input_ids, word_emb, pos_emb, type_emb, gamma, beta):
    return _run(input_ids, word_emb, pos_emb, type_emb, gamma, beta)


# speedup vs baseline: 1.0784x; 1.0250x over previous
import functools

import jax
import jax.numpy as jnp
from jax import lax
from jax.experimental import pallas as pl
from jax.experimental.pallas import tpu as pltpu
from jax.experimental.pallas import tpu_sc as plsc

_B = 4
_S = 2048
_D = 768
_EPS = 1e-12

_N_TOK = _B * _S
_PCHUNKS = (1024, 512, 512)
_PSTARTS = (0, 1024, 1536)
_NW = 32
_WSEG = _NW // _B
_K = 64


def _gather_body(pstart, plen, ids_hbm, wemb_hbm, out_hbm,
                 idx0, idx1, row0, row1, sg0, sg1, ss0, ss1):
    cid = lax.axis_index("c")
    sid = lax.axis_index("s")
    w = sid * 2 + cid
    seg = w // _WSEG
    tpw = plen // _WSEG
    ngr = tpw // _K
    off = (w % _WSEG) * tpw
    pbase = pstart + off
    obase = seg * plen + off
    idx = (idx0, idx1)
    row = (row0, row1)
    sg = (sg0, sg1)
    ss = (ss0, ss1)

    pltpu.sync_copy(ids_hbm.at[seg, pl.ds(pbase, _K)], idx0)
    pltpu.async_copy(wemb_hbm.at[idx0], row0, sg0)
    for c in range(ngr):
        b = c & 1
        if c + 1 < ngr:
            pltpu.sync_copy(ids_hbm.at[seg, pl.ds(pbase + (c + 1) * _K, _K)],
                            idx[1 - b])
            if c >= 1:
                pltpu.make_async_copy(
                    row[1 - b], out_hbm.at[pl.ds(obase + (c - 1) * _K, _K)],
                    ss[1 - b]).wait()
            pltpu.async_copy(wemb_hbm.at[idx[1 - b]], row[1 - b], sg[1 - b])
        pltpu.make_async_copy(wemb_hbm.at[idx[b]], row[b], sg[b]).wait()
        pltpu.async_copy(row[b], out_hbm.at[pl.ds(obase + c * _K, _K)], ss[b])
    for c in range(max(0, ngr - 2), ngr):
        b = c & 1
        pltpu.make_async_copy(
            row[b], out_hbm.at[pl.ds(obase + c * _K, _K)], ss[b]).wait()


def _sc_gather(ids, wemb, chunk):
    pstart = _PSTARTS[chunk]
    plen = _PCHUNKS[chunk]
    mesh = plsc.VectorSubcoreMesh(core_axis_name="c", subcore_axis_name="s")
    f = functools.partial(
        pl.kernel,
        mesh=mesh,
        compiler_params=pltpu.CompilerParams(needs_layout_passes=False),
        out_type=jax.ShapeDtypeStruct((_B * plen, _D), jnp.float32),
        scratch_types=[
            pltpu.VMEM((_K,), jnp.int32),
            pltpu.VMEM((_K,), jnp.int32),
            pltpu.VMEM((_K, _D), jnp.float32),
            pltpu.VMEM((_K, _D), jnp.float32),
            pltpu.SemaphoreType.DMA,
            pltpu.SemaphoreType.DMA,
            pltpu.SemaphoreType.DMA,
            pltpu.SemaphoreType.DMA,
        ],
    )(functools.partial(_gather_body, pstart, plen))
    return f(ids, wemb)


def _ln_body(rows_ref, pos_ref, typ_ref, gam_ref, bet_ref, out_ref):
    x = rows_ref[...] + pos_ref[...] + typ_ref[...]
    mean = jnp.mean(x, axis=1, keepdims=True)
    xc = x - mean
    var = jnp.mean(xc * xc, axis=1, keepdims=True)
    y = xc * lax.rsqrt(var + _EPS)
    out_ref[...] = (y * gam_ref[...] + bet_ref[...]).astype(jnp.bfloat16)


def _ln_body_alias(prev_ref, rows_ref, pos_ref, typ_ref, gam_ref, bet_ref,
                   out_ref):
    del prev_ref
    _ln_body(rows_ref, pos_ref, typ_ref, gam_ref, bet_ref, out_ref)


def _ln_specs(chunk):
    plen = _PCHUNKS[chunk]
    poff = _PSTARTS[chunk] // plen
    return [
        pl.BlockSpec((plen, _D), lambda b: (b, 0)),
        pl.BlockSpec((plen, _D), lambda b: (poff, 0)),
        pl.BlockSpec((1, _D), lambda b: (0, 0)),
        pl.BlockSpec((1, _D), lambda b: (0, 0)),
        pl.BlockSpec((1, _D), lambda b: (0, 0)),
    ]


def _ln_out_spec(chunk):
    plen = _PCHUNKS[chunk]
    boff = _PSTARTS[chunk] // plen
    nblk = _S // plen
    return pl.BlockSpec((plen, _D), lambda b: (b * nblk + boff, 0))


def _tc_ln_chunk(prev, rows, pos, typ0, gam2, bet2, chunk):
    out_shape = jax.ShapeDtypeStruct((_N_TOK, _D), jnp.bfloat16)
    if prev is None:
        return pl.pallas_call(
            _ln_body,
            grid=(_B,),
            in_specs=_ln_specs(chunk),
            out_specs=_ln_out_spec(chunk),
            out_shape=out_shape,
        )(rows, pos, typ0, gam2, bet2)
    return pl.pallas_call(
        _ln_body_alias,
        grid=(_B,),
        in_specs=[pl.BlockSpec(memory_space=pl.ANY)] + _ln_specs(chunk),
        out_specs=_ln_out_spec(chunk),
        out_shape=out_shape,
        input_output_aliases={0: 0},
    )(prev, rows, pos, typ0, gam2, bet2)


@jax.jit
def _run(input_ids, wemb, pemb, temb, gam, bet):
    ids = input_ids.astype(jnp.int32)
    typ0 = temb[0:1]
    gam2 = gam.reshape(1, _D)
    bet2 = bet.reshape(1, _D)
    rows = [_sc_gather(ids, wemb, c) for c in range(len(_PCHUNKS))]
    buf = None
    for c in range(len(_PCHUNKS)):
        buf = _tc_ln_chunk(buf, rows[c], pemb, typ0, gam2, bet2, c)
    return buf.reshape(_B, _S, _D)


def kernel(input_ids, word_emb, pos_emb, type_emb, gamma, beta):
    return _run(input_ids, word_emb, pos_emb, type_emb, gamma, beta)


# speedup vs baseline: 1.1354x; 1.0528x over previous
import functools

import jax
import jax.numpy as jnp
from jax import lax
from jax.experimental import pallas as pl
from jax.experimental.pallas import tpu as pltpu
from jax.experimental.pallas import tpu_sc as plsc

_B = 4
_S = 2048
_D = 768
_EPS = 1e-12

_N_TOK = _B * _S
_NHALF = _N_TOK // 2
_PHALF = _S // 2
_NW = 32
_WSEG = _NW // _B
_TPW = _PHALF // _WSEG
_K = 64
_NCH = _TPW // _K


def _gather_body(half, ids_hbm, wemb_hbm, out_hbm,
                 idx0, idx1, row0, row1, sg0, sg1, ss0, ss1):
    cid = lax.axis_index("c")
    sid = lax.axis_index("s")
    w = sid * 2 + cid
    seg = w // _WSEG
    off = (w % _WSEG) * _TPW
    pbase = half * _PHALF + off
    obase = seg * _PHALF + off
    idx = (idx0, idx1)
    row = (row0, row1)
    sg = (sg0, sg1)
    ss = (ss0, ss1)

    pltpu.sync_copy(ids_hbm.at[seg, pl.ds(pbase, _K)], idx0)
    pltpu.async_copy(wemb_hbm.at[idx0], row0, sg0)
    for c in range(_NCH):
        b = c & 1
        if c + 1 < _NCH:
            pltpu.sync_copy(ids_hbm.at[seg, pl.ds(pbase + (c + 1) * _K, _K)],
                            idx[1 - b])
            if c >= 1:
                pltpu.make_async_copy(
                    row[1 - b], out_hbm.at[pl.ds(obase + (c - 1) * _K, _K)],
                    ss[1 - b]).wait()
            pltpu.async_copy(wemb_hbm.at[idx[1 - b]], row[1 - b], sg[1 - b])
        pltpu.make_async_copy(wemb_hbm.at[idx[b]], row[b], sg[b]).wait()
        pltpu.async_copy(row[b], out_hbm.at[pl.ds(obase + c * _K, _K)], ss[b])
    for c in range(max(0, _NCH - 2), _NCH):
        b = c & 1
        pltpu.make_async_copy(
            row[b], out_hbm.at[pl.ds(obase + c * _K, _K)], ss[b]).wait()


def _sc_gather(ids, wemb, half):
    mesh = plsc.VectorSubcoreMesh(core_axis_name="c", subcore_axis_name="s")
    f = functools.partial(
        pl.kernel,
        mesh=mesh,
        compiler_params=pltpu.CompilerParams(needs_layout_passes=False),
        out_type=jax.ShapeDtypeStruct((_NHALF, _D), jnp.float32),
        scratch_types=[
            pltpu.VMEM((_K,), jnp.int32),
            pltpu.VMEM((_K,), jnp.int32),
            pltpu.VMEM((_K, _D), jnp.float32),
            pltpu.VMEM((_K, _D), jnp.float32),
            pltpu.SemaphoreType.DMA,
            pltpu.SemaphoreType.DMA,
            pltpu.SemaphoreType.DMA,
            pltpu.SemaphoreType.DMA,
        ],
    )(functools.partial(_gather_body, half))
    return f(ids, wemb)


def _ln_body(rows_ref, pos_ref, typ_ref, gam_ref, bet_ref, out_ref):
    x = rows_ref[...] + pos_ref[...] + typ_ref[...]
    mean = jnp.mean(x, axis=1, keepdims=True)
    xc = x - mean
    var = jnp.mean(xc * xc, axis=1, keepdims=True)
    y = xc * lax.rsqrt(var + _EPS)
    out_ref[...] = (y * gam_ref[...] + bet_ref[...]).astype(jnp.bfloat16)


def _ln_body_alias(prev_ref, rows_ref, pos_ref, typ_ref, gam_ref, bet_ref,
                   out_ref):
    del prev_ref
    _ln_body(rows_ref, pos_ref, typ_ref, gam_ref, bet_ref, out_ref)


def _ln_specs(half):
    return [
        pl.BlockSpec((_PHALF, _D), lambda b: (b, 0)),
        pl.BlockSpec((_PHALF, _D), lambda b: (half, 0)),
        pl.BlockSpec((1, _D), lambda b: (0, 0)),
        pl.BlockSpec((1, _D), lambda b: (0, 0)),
        pl.BlockSpec((1, _D), lambda b: (0, 0)),
    ]


def _ln_out_spec(half):
    return pl.BlockSpec((_PHALF, _D), lambda b: (b * 2 + half, 0))


def _tc_ln_half(prev, rows, pos, typ0, gam2, bet2, half):
    out_shape = jax.ShapeDtypeStruct((_N_TOK, _D), jnp.bfloat16)
    if prev is None:
        return pl.pallas_call(
            _ln_body,
            grid=(_B,),
            in_specs=_ln_specs(half),
            out_specs=_ln_out_spec(half),
            out_shape=out_shape,
        )(rows, pos, typ0, gam2, bet2)
    return pl.pallas_call(
        _ln_body_alias,
        grid=(_B,),
        in_specs=[pl.BlockSpec(memory_space=pl.ANY)] + _ln_specs(half),
        out_specs=_ln_out_spec(half),
        out_shape=out_shape,
        input_output_aliases={0: 0},
    )(prev, rows, pos, typ0, gam2, bet2)


@jax.jit
def _run(input_ids, wemb, pemb, temb, gam, bet):
    ids = input_ids.astype(jnp.int32)
    typ0 = temb[0:1]
    gam2 = gam.reshape(1, _D)
    bet2 = bet.reshape(1, _D)
    rows_lo = _sc_gather(ids, wemb, 0)
    rows_hi = _sc_gather(ids, wemb, 1)
    buf = _tc_ln_half(None, rows_lo, pemb, typ0, gam2, bet2, 0)
    out = _tc_ln_half(buf, rows_hi, pemb, typ0, gam2, bet2, 1)
    return out.reshape(_B, _S, _D)


def kernel(input_ids, word_emb, pos_emb, type_emb, gamma, beta):
    return _run(input_ids, word_emb, pos_emb, type_emb, gamma, beta)


# speedup vs baseline: 1.1424x; 1.0062x over previous
import functools

import jax
import jax.numpy as jnp
from jax import lax
from jax.experimental import pallas as pl
from jax.experimental.pallas import tpu as pltpu
from jax.experimental.pallas import tpu_sc as plsc

_B = 4
_S = 2048
_D = 768
_EPS = 1e-12

_N_TOK = _B * _S
_NW = 32
_WSEG = _NW // _B
_TPW = _S // _WSEG
_K = 64
_NCH = _TPW // _K


def _gather_body(ids_hbm, wemb_hbm, out_hbm,
                 idx0, idx1, row0, row1, sg0, sg1, ss0, ss1):
    cid = lax.axis_index("c")
    sid = lax.axis_index("s")
    w = sid * 2 + cid
    seg = w // _WSEG
    off = (w % _WSEG) * _TPW
    obase = seg * _S + off
    idx = (idx0, idx1)
    row = (row0, row1)
    sg = (sg0, sg1)
    ss = (ss0, ss1)

    pltpu.sync_copy(ids_hbm.at[seg, pl.ds(off, _K)], idx0)
    pltpu.async_copy(wemb_hbm.at[idx0], row0, sg0)
    for c in range(_NCH):
        b = c & 1
        if c + 1 < _NCH:
            pltpu.sync_copy(ids_hbm.at[seg, pl.ds(off + (c + 1) * _K, _K)],
                            idx[1 - b])
            if c >= 1:
                pltpu.make_async_copy(
                    row[1 - b], out_hbm.at[pl.ds(obase + (c - 1) * _K, _K)],
                    ss[1 - b]).wait()
            pltpu.async_copy(wemb_hbm.at[idx[1 - b]], row[1 - b], sg[1 - b])
        pltpu.make_async_copy(wemb_hbm.at[idx[b]], row[b], sg[b]).wait()
        pltpu.async_copy(row[b], out_hbm.at[pl.ds(obase + c * _K, _K)], ss[b])
    for c in range(max(0, _NCH - 2), _NCH):
        b = c & 1
        pltpu.make_async_copy(
            row[b], out_hbm.at[pl.ds(obase + c * _K, _K)], ss[b]).wait()


def _sc_gather(ids, wemb):
    mesh = plsc.VectorSubcoreMesh(core_axis_name="c", subcore_axis_name="s")
    f = pl.kernel(
        _gather_body,
        mesh=mesh,
        compiler_params=pltpu.CompilerParams(needs_layout_passes=False),
        out_type=jax.ShapeDtypeStruct((_N_TOK, _D), jnp.float32),
        scratch_types=[
            pltpu.VMEM((_K,), jnp.int32),
            pltpu.VMEM((_K,), jnp.int32),
            pltpu.VMEM((_K, _D), jnp.float32),
            pltpu.VMEM((_K, _D), jnp.float32),
            pltpu.SemaphoreType.DMA,
            pltpu.SemaphoreType.DMA,
            pltpu.SemaphoreType.DMA,
            pltpu.SemaphoreType.DMA,
        ],
    )
    return f(ids, wemb)


def _ln_body(rows_ref, pos_ref, typ_ref, gam_ref, bet_ref, out_ref):
    x = rows_ref[...] + pos_ref[...] + typ_ref[...]
    mean = jnp.mean(x, axis=1, keepdims=True)
    xc = x - mean
    var = jnp.mean(xc * xc, axis=1, keepdims=True)
    y = xc * lax.rsqrt(var + _EPS)
    out_ref[...] = (y * gam_ref[...] + bet_ref[...]).astype(jnp.bfloat16)


@jax.jit
def _run(input_ids, wemb, pemb, temb, gam, bet):
    ids = input_ids.astype(jnp.int32)
    typ0 = temb[0:1]
    gam2 = gam.reshape(1, _D)
    bet2 = bet.reshape(1, _D)
    rows = _sc_gather(ids, wemb)
    out = pl.pallas_call(
        _ln_body,
        grid=(_B,),
        in_specs=[
            pl.BlockSpec((_S, _D), lambda b: (b, 0)),
            pl.BlockSpec((_S, _D), lambda b: (0, 0)),
            pl.BlockSpec((1, _D), lambda b: (0, 0)),
            pl.BlockSpec((1, _D), lambda b: (0, 0)),
            pl.BlockSpec((1, _D), lambda b: (0, 0)),
        ],
        out_specs=pl.BlockSpec((_S, _D), lambda b: (b, 0)),
        out_shape=jax.ShapeDtypeStruct((_N_TOK, _D), jnp.bfloat16),
    )(rows, pemb, typ0, gam2, bet2)
    return out.reshape(_B, _S, _D)


def kernel(input_ids, word_emb, pos_emb, type_emb, gamma, beta):
    return _run(input_ids, word_emb, pos_emb, type_emb, gamma, beta)
